# baseline (speedup 1.0000x reference)
import functools

import numpy as np
import jax
import jax.numpy as jnp
from jax import lax
from jax.experimental import pallas as pl
from jax.experimental.pallas import tpu as pltpu
from jax.experimental.pallas import tpu_sc as plsc

N_NODES = 10000
N_EDGES = 320000
F = 128
FP = F // 2
G = 16

NC = 2
NS = 16
NW = NC * NS
EDGES_PER_W = N_EDGES // NW
CHUNK = 80
NCHUNKS = EDGES_PER_W // CHUNK
NRING = 3
NMAIN = NCHUNKS - 2
ROWS_PER_TILE = 624
ROWS_TAIL = N_NODES - NS * ROWS_PER_TILE
NVEC = F // 16

PERM = np.concatenate(
    [np.concatenate([np.arange(32 * q, 32 * q + 32, 2),
                     np.arange(32 * q + 1, 32 * q + 32, 2)])
     for q in range(4)])
_MASK_HI = jnp.int32(-65536)


def _bcast_lane(v, i):
    idx = lax.full((16, 1), i, jnp.int32)
    return lax.gather(
        v, idx,
        lax.GatherDimensionNumbers(
            offset_dims=(), collapsed_slice_dims=(0,), start_index_map=(0,)),
        (1,), mode=lax.GatherScatterMode.PROMISE_IN_BOUNDS)


def _sc_agg_body(xi_hbm, ei_hbm, attr_hbm, out_hbm,
                 acc, srcv, dstv, attrv, rowsi, rowsf,
                 sem_g, sem_s, sem_ia, sem_id):
    c = lax.axis_index("c")
    s = lax.axis_index("s")
    wid = s * NC + c
    ebase = wid * EDGES_PER_W
    rbase = s * ROWS_PER_TILE

    def issue_ia(k, b):
        base = ebase + k * CHUNK
        pltpu.async_copy(ei_hbm.at[pl.ds(base, CHUNK)], srcv[b], sem_ia[b])
        pltpu.async_copy(attr_hbm.at[pl.ds(base, CHUNK)], attrv[b], sem_ia[b])

    def wait_ia(b):
        pltpu.make_async_copy(ei_hbm.at[pl.ds(0, CHUNK)], srcv[b],
                              sem_ia[b]).wait()
        pltpu.make_async_copy(attr_hbm.at[pl.ds(0, CHUNK)], attrv[b],
                              sem_ia[b]).wait()

    def issue_id(k, b):
        base = ebase + k * CHUNK
        pltpu.async_copy(ei_hbm.at[pl.ds(N_EDGES + base, CHUNK)],
                         dstv[b], sem_id[b])

    def wait_id(b):
        pltpu.make_async_copy(ei_hbm.at[pl.ds(0, CHUNK)], dstv[b],
                              sem_id[b]).wait()

    def issue_gather(b):
        pltpu.async_copy(xi_hbm.at[srcv[b]], rowsi[b], sem_g[b])

    def wait_gather(b):
        pltpu.make_async_copy(xi_hbm.at[srcv[b]], rowsi[b], sem_g[b]).wait()

    def issue_scatter(b):
        pltpu.async_copy(rowsf[b], acc.at[dstv[b]], sem_s[b], add=True)

    def wait_scatter(b):
        pltpu.make_async_copy(rowsf[b], acc.at[dstv[b]], sem_s[b]).wait()

    def multiply(b):
        def _group(g, carry):
            av = attrv[b][pl.ds(g * 16, 16)]
            for i in range(16):
                sc = _bcast_lane(av, i)
                e = g * 16 + i
                for q in range(FP // 16):
                    vi = rowsi[b][e, pl.ds(q * 16, 16)]
                    lo = lax.bitcast_convert_type(
                        lax.shift_left(vi, 16), jnp.float32)
                    hi = lax.bitcast_convert_type(vi & _MASK_HI, jnp.float32)
                    rowsf[b][e, pl.ds(q * 32, 16)] = lo * sc
                    rowsf[b][e, pl.ds(q * 32 + 16, 16)] = hi * sc
            return carry
        lax.fori_loop(0, CHUNK // 16, _group, 0)

    issue_ia(0, 0)
    issue_ia(1, 1)
    issue_id(0, 0)

    def _zero_row(r, carry):
        for j in range(NVEC):
            rowsf[2][r, pl.ds(j * 16, 16)] = jnp.zeros((16,), jnp.float32)
        return carry
    lax.fori_loop(0, CHUNK, _zero_row, 0)
    for t in range(ROWS_PER_TILE // CHUNK):
        pltpu.sync_copy(rowsf[2], acc.at[pl.ds(rbase + t * CHUNK, CHUNK)])
    _rem = ROWS_PER_TILE % CHUNK
    if _rem:
        pltpu.sync_copy(
            rowsf[2].at[pl.ds(0, _rem)],
            acc.at[pl.ds(rbase + (ROWS_PER_TILE // CHUNK) * CHUNK, _rem)])

    @pl.when(s == 0)
    def _zero_tail():
        pltpu.sync_copy(rowsf[2].at[pl.ds(0, ROWS_TAIL)],
                        acc.at[pl.ds(NS * ROWS_PER_TILE, ROWS_TAIL)])

    wait_ia(0)
    issue_gather(0)
    wait_ia(1)
    issue_gather(1)
    issue_ia(2, 2)
    plsc.subcore_barrier()

    def _super(it, carry):
        for r in range(NRING):
            k = it * NRING + r
            @pl.when(k >= 2)
            def _w():
                wait_scatter((r + 1) % NRING)
            @pl.when(k + 1 < NCHUNKS)
            def _b2():
                issue_id(k + 1, (r + 1) % NRING)
            wait_ia((r + 2) % NRING)
            issue_gather((r + 2) % NRING)
            wait_gather(r)
            multiply(r)
            wait_id(r)
            issue_scatter(r)
            @pl.when(k + 3 < NCHUNKS)
            def _b1():
                issue_ia(k + 3, r)
        return carry
    lax.fori_loop(0, NMAIN // NRING, _super, 0)

    wait_scatter(1)
    issue_id(124, 1)
    wait_gather(0)
    multiply(0)
    wait_id(0)
    issue_scatter(0)
    wait_scatter(2)
    wait_gather(1)
    multiply(1)
    wait_id(1)
    issue_scatter(1)
    wait_scatter(0)
    wait_scatter(1)

    plsc.subcore_barrier()
    pltpu.sync_copy(acc.at[pl.ds(rbase, ROWS_PER_TILE)],
                    out_hbm.at[c, pl.ds(rbase, ROWS_PER_TILE)])

    @pl.when(s == 0)
    def _out_tail():
        pltpu.sync_copy(acc.at[pl.ds(NS * ROWS_PER_TILE, ROWS_TAIL)],
                        out_hbm.at[c, pl.ds(NS * ROWS_PER_TILE, ROWS_TAIL)])


@jax.jit
def _sc_agg(xi, ei, attr):
    mesh = plsc.VectorSubcoreMesh(core_axis_name="c", subcore_axis_name="s")
    return pl.kernel(
        _sc_agg_body,
        out_type=jax.ShapeDtypeStruct((NC, N_NODES, F), jnp.float32),
        mesh=mesh,
        compiler_params=pltpu.CompilerParams(use_tc_tiling_on_sc=False),
        scratch_types=[
            pltpu.VMEM_SHARED((N_NODES, F), jnp.float32),
            [pltpu.VMEM((CHUNK,), jnp.int32) for _ in range(NRING)],
            [pltpu.VMEM((CHUNK,), jnp.int32) for _ in range(NRING)],
            [pltpu.VMEM((CHUNK,), jnp.float32) for _ in range(NRING)],
            [pltpu.VMEM((CHUNK, FP), jnp.int32) for _ in range(NRING)],
            [pltpu.VMEM((CHUNK, F), jnp.float32) for _ in range(NRING)],
            [pltpu.SemaphoreType.DMA for _ in range(NRING)],
            [pltpu.SemaphoreType.DMA for _ in range(NRING)],
            [pltpu.SemaphoreType.DMA for _ in range(NRING)],
            [pltpu.SemaphoreType.DMA for _ in range(NRING)],
        ],
    )(xi, ei, attr)


ROWBLK = 2000
NBLK = N_NODES // ROWBLK


def _tc_body(se_ref, agg_ref, u_ref, wt, b2, k_ref, q_ref):
    xa = agg_ref[0] + agg_ref[1]
    ri = (lax.broadcasted_iota(jnp.int32, (ROWBLK, G), 0)
          + pl.program_id(0) * ROWBLK)
    oh = ((ri >= se_ref[0:1, :]) & (ri < se_ref[1:2, :])).astype(jnp.float32)
    hp = lax.Precision.HIGHEST
    uw = jnp.dot(u_ref[...], wt[F:, :], precision=hp)
    kq = (jnp.dot(xa, wt[:F, :], precision=hp)
          + jnp.dot(oh, uw, precision=hp) + b2[...])
    k_ref[...] = kq[:, :F]
    q_ref[...] = kq[:, F:]


@jax.jit
def _tc_linear(se, agg, u, wt, b2):
    full = lambda *shape: pl.BlockSpec(shape, lambda i: tuple(0 for _ in shape))
    return pl.pallas_call(
        _tc_body,
        grid=(NBLK,),
        in_specs=[
            full(2, G),
            pl.BlockSpec((NC, ROWBLK, F), lambda i: (0, i, 0)),
            full(G, F),
            full(2 * F, 2 * F),
            full(1, 2 * F),
        ],
        out_specs=[
            pl.BlockSpec((ROWBLK, F), lambda i: (i, 0)),
            pl.BlockSpec((ROWBLK, F), lambda i: (i, 0)),
        ],
        out_shape=[
            jax.ShapeDtypeStruct((N_NODES, F), jnp.float32),
            jax.ShapeDtypeStruct((N_NODES, F), jnp.float32),
        ],
    )(se, agg, u, wt, b2)


def kernel(x, edge_index, edge_attr, u, batch, WK, bK, WQ, bQ):
    xi = lax.bitcast_convert_type(
        x.astype(jnp.bfloat16).reshape(N_NODES, FP, 2), jnp.int32)
    ei = edge_index.astype(jnp.int32).reshape(2 * N_EDGES)
    attr = edge_attr.T.reshape(N_EDGES)
    agg = _sc_agg(xi, ei, attr)
    b32 = batch.astype(jnp.int32)
    gids = jnp.arange(G, dtype=jnp.int32)
    starts = jnp.sum(b32[None, :] < gids[:, None], axis=1, dtype=jnp.int32)
    ends = jnp.concatenate([starts[1:], jnp.array([N_NODES], jnp.int32)])
    se = jnp.stack([starts, ends])
    wx = jnp.concatenate([WK, WQ], axis=0).T[:F][PERM]
    wu = jnp.concatenate([WK, WQ], axis=0).T[F:]
    wt = jnp.concatenate([wx, wu], axis=0)
    b2 = jnp.concatenate([bK, bQ]).reshape(1, 2 * F)
    K, Q = _tc_linear(se, agg, u, wt, b2)
    return K, Q
